# baseline (speedup 1.0000x reference)
import functools

import jax
import jax.numpy as jnp
from jax import lax
from jax.experimental import pallas as pl
from jax.experimental.pallas import tpu as pltpu
from jax.experimental.pallas import tpu_sc as plsc

H = 1024
H2 = H // 2
F = 2048
E = 8
K = 2
T = 2048

TB = 256
NTB = T // TB
BLK = 256
P = 6144
NB = P // BLK

_NTILES = 32
_RPT = P // _NTILES
_GCH = _RPT // 2
_TPT = T // _NTILES
_CCH = _TPT // 2

_mesh = functools.partial(
    plsc.VectorSubcoreMesh, core_axis_name="c", subcore_axis_name="s")
_sc_params = pltpu.CompilerParams(needs_layout_passes=False)


def _router_body(x_ref, wgt_ref, e1_ref, e2_ref, r1_ref, r2_ref,
                 wa_ref, wb_ref, cnt_ref, carry_ref):
    tb = pl.program_id(0)
    logits = lax.dot_general(
        x_ref[...].astype(jnp.bfloat16), wgt_ref[...].astype(jnp.bfloat16),
        (((1,), (0,)), ((), ())),
        preferred_element_type=jnp.float32)
    lane = lax.broadcasted_iota(jnp.int32, (TB, E), 1)
    big = jnp.int32(E)
    l1 = jnp.max(logits, axis=1, keepdims=True)
    i1 = jnp.min(jnp.where(logits == l1, lane, big), axis=1, keepdims=True)
    masked = jnp.where(lane == i1, -jnp.inf, logits)
    l2 = jnp.max(masked, axis=1, keepdims=True)
    i2 = jnp.min(jnp.where(masked == l2, lane, big), axis=1, keepdims=True)
    wb = 1.0 / (1.0 + jnp.exp(l1 - l2))
    wa = 1.0 - wb

    mask = ((lane == i1) | (lane == i2)).astype(jnp.bfloat16)
    row_i = lax.broadcasted_iota(jnp.int32, (TB, TB), 0)
    col_i = lax.broadcasted_iota(jnp.int32, (TB, TB), 1)
    tri = (col_i < row_i).astype(jnp.bfloat16)
    rank = lax.dot_general(tri, mask, (((1,), (0,)), ((), ())),
                           preferred_element_type=jnp.float32)

    @pl.when(tb == 0)
    def _():
        carry_ref[...] = jnp.zeros_like(carry_ref)

    carry = carry_ref[0:1, 0:E]
    rank = rank + carry
    new_carry = carry + jnp.sum(mask.astype(jnp.float32), axis=0,
                                keepdims=True)
    carry_ref[0:1, 0:E] = new_carry

    e1_ref[...] = i1
    e2_ref[...] = i2
    r1_ref[...] = jnp.sum(jnp.where(lane == i1, rank, 0.0), axis=1,
                          keepdims=True).astype(jnp.int32)
    r2_ref[...] = jnp.sum(jnp.where(lane == i2, rank, 0.0), axis=1,
                          keepdims=True).astype(jnp.int32)
    wa_ref[...] = wa
    wb_ref[...] = wb

    @pl.when(tb == NTB - 1)
    def _():
        cnt_ref[...] = jnp.broadcast_to(new_carry, (E, E))


def _router(x, Wg):
    out_shapes = [
        jax.ShapeDtypeStruct((T, 1), jnp.int32),
        jax.ShapeDtypeStruct((T, 1), jnp.int32),
        jax.ShapeDtypeStruct((T, 1), jnp.int32),
        jax.ShapeDtypeStruct((T, 1), jnp.int32),
        jax.ShapeDtypeStruct((T, 1), jnp.float32),
        jax.ShapeDtypeStruct((T, 1), jnp.float32),
        jax.ShapeDtypeStruct((E, E), jnp.float32),
    ]
    tok_spec = lambda: pl.BlockSpec((TB, 1), lambda tb: (tb, 0))
    return pl.pallas_call(
        _router_body,
        grid=(NTB,),
        in_specs=[
            pl.BlockSpec((TB, H), lambda tb: (tb, 0)),
            pl.BlockSpec((H, E), lambda tb: (0, 0)),
        ],
        out_specs=[
            tok_spec(), tok_spec(), tok_spec(), tok_spec(),
            tok_spec(), tok_spec(),
            pl.BlockSpec((E, E), lambda tb: (0, 0)),
        ],
        out_shape=out_shapes,
        scratch_shapes=[pltpu.VMEM((8, 128), jnp.float32)],
    )(x, Wg.T)


def _scatter_body(e1_ref, e2_ref, r1_ref, r2_ref, wa_ref, wb_ref, off_ref,
                  st_ref, cwp_ref, pos0_ref, pos1_ref,
                  e1_v, e2_v, r1_v, r2_v, wa_v, wb_v, off_v,
                  st_v, cw_v, p0_v, p1_v):
    wid = lax.axis_index("s") * 2 + lax.axis_index("c")

    @pl.when(wid == 0)
    def _():
        pltpu.sync_copy(e1_ref, e1_v)
        pltpu.sync_copy(e2_ref, e2_v)
        pltpu.sync_copy(r1_ref, r1_v)
        pltpu.sync_copy(r2_ref, r2_v)
        pltpu.sync_copy(wa_ref, wa_v)
        pltpu.sync_copy(wb_ref, wb_v)
        pltpu.sync_copy(off_ref, off_v)

        def zero_body(i, _):
            st_v[pl.ds(i * 16, 16)] = jnp.zeros((16,), jnp.int32)
            cw_v[pl.ds(i * 16, 16)] = jnp.zeros((16,), jnp.float32)
            return 0
        lax.fori_loop(0, P // 16, zero_body, 0)

        def chunk_body(j, _):
            sl = pl.ds(j * 16, 16)
            tok = lax.iota(jnp.int32, 16) + j * 16
            pos = plsc.load_gather(off_v, [e1_v[sl]]) + r1_v[sl]
            plsc.store_scatter(st_v, [pos], tok)
            plsc.store_scatter(cw_v, [pos], wa_v[sl])
            p0_v[sl] = pos
            pos2 = plsc.load_gather(off_v, [e2_v[sl]]) + r2_v[sl]
            plsc.store_scatter(st_v, [pos2], tok)
            plsc.store_scatter(cw_v, [pos2], wb_v[sl])
            p1_v[sl] = pos2
            return 0
        lax.fori_loop(0, T // 16, chunk_body, 0)

        pltpu.sync_copy(st_v, st_ref)
        pltpu.sync_copy(cw_v, cwp_ref)
        pltpu.sync_copy(p0_v, pos0_ref)
        pltpu.sync_copy(p1_v, pos1_ref)


def _scatter(e1, e2, r1, r2, wa, wb, off16):
    return pl.kernel(
        _scatter_body,
        mesh=_mesh(),
        compiler_params=_sc_params,
        out_type=[
            jax.ShapeDtypeStruct((P,), jnp.int32),
            jax.ShapeDtypeStruct((P,), jnp.float32),
            jax.ShapeDtypeStruct((T,), jnp.int32),
            jax.ShapeDtypeStruct((T,), jnp.int32),
        ],
        scratch_types=[
            pltpu.VMEM((T,), jnp.int32), pltpu.VMEM((T,), jnp.int32),
            pltpu.VMEM((T,), jnp.int32), pltpu.VMEM((T,), jnp.int32),
            pltpu.VMEM((T,), jnp.float32), pltpu.VMEM((T,), jnp.float32),
            pltpu.VMEM((16,), jnp.int32),
            pltpu.VMEM((P,), jnp.int32), pltpu.VMEM((P,), jnp.float32),
            pltpu.VMEM((T,), jnp.int32), pltpu.VMEM((T,), jnp.int32),
        ],
    )(e1, e2, r1, r2, wa, wb, off16)


def _gather_body(xb_ref, st_ref, xs_ref, idx_v, rows_v, sem):
    wid = lax.axis_index("s") * 2 + lax.axis_index("c")
    for c in range(_RPT // _GCH):
        base = wid * _RPT + c * _GCH
        pltpu.sync_copy(st_ref.at[pl.ds(base, _GCH)], idx_v)
        pltpu.async_copy(xb_ref.at[idx_v], rows_v, sem).wait()
        pltpu.sync_copy(rows_v, xs_ref.at[pl.ds(base, _GCH), :])


def _gather(xb32, st):
    return pl.kernel(
        _gather_body,
        mesh=_mesh(),
        compiler_params=_sc_params,
        out_type=jax.ShapeDtypeStruct((P, H2), jnp.int32),
        scratch_types=[
            pltpu.VMEM((_GCH,), jnp.int32),
            pltpu.VMEM((_GCH, H2), jnp.int32),
            pltpu.SemaphoreType.DMA,
        ],
    )(xb32, st)


def _ffn_body(be_ref, xs_ref, w1_ref, w3_ref, w2_ref, cwp_ref, ys_ref):
    xb = xs_ref[...]
    h = jnp.dot(xb, w1_ref[0], preferred_element_type=jnp.float32)
    g = jnp.dot(xb, w3_ref[0], preferred_element_type=jnp.float32)
    a = (h * lax.logistic(h) * g).astype(jnp.bfloat16)
    y = jnp.dot(a, w2_ref[0], preferred_element_type=jnp.float32)
    ys_ref[...] = y * cwp_ref[...]


def _ffn(xs, be, W1T, W3T, W2T, cwp):
    grid_spec = pltpu.PrefetchScalarGridSpec(
        num_scalar_prefetch=1,
        grid=(NB,),
        in_specs=[
            pl.BlockSpec((BLK, H), lambda b, be: (b, 0)),
            pl.BlockSpec((1, H, F), lambda b, be: (be[b], 0, 0)),
            pl.BlockSpec((1, H, F), lambda b, be: (be[b], 0, 0)),
            pl.BlockSpec((1, F, H), lambda b, be: (be[b], 0, 0)),
            pl.BlockSpec((BLK, 1), lambda b, be: (b, 0)),
        ],
        out_specs=pl.BlockSpec((BLK, H), lambda b, be: (b, 0)),
    )
    return pl.pallas_call(
        _ffn_body,
        grid_spec=grid_spec,
        out_shape=jax.ShapeDtypeStruct((P, H), jnp.float32),
        compiler_params=pltpu.CompilerParams(
            dimension_semantics=("arbitrary",)),
    )(be, xs, W1T, W3T, W2T, cwp)


def _combine_body(ys_ref, pos0_ref, pos1_ref, out_ref,
                  idxa_v, idxb_v, rows_a, rows_b, obuf, sema, semb):
    wid = lax.axis_index("s") * 2 + lax.axis_index("c")
    for c in range(_TPT // _CCH):
        base = wid * _TPT + c * _CCH
        pltpu.sync_copy(pos0_ref.at[pl.ds(base, _CCH)], idxa_v)
        pltpu.sync_copy(pos1_ref.at[pl.ds(base, _CCH)], idxb_v)
        cpa = pltpu.async_copy(ys_ref.at[idxa_v], rows_a, sema)
        cpb = pltpu.async_copy(ys_ref.at[idxb_v], rows_b, semb)
        cpa.wait()
        cpb.wait()

        def tok_body(i, _):
            for q in range(H // 16):
                sl = pl.ds(q * 16, 16)
                obuf[i, sl] = rows_a[i, sl] + rows_b[i, sl]
            return 0
        lax.fori_loop(0, _CCH, tok_body, 0)

        pltpu.sync_copy(obuf, out_ref.at[pl.ds(base, _CCH), :])


def _combine(ys, pos0, pos1):
    return pl.kernel(
        _combine_body,
        mesh=_mesh(),
        compiler_params=_sc_params,
        out_type=jax.ShapeDtypeStruct((T, H), jnp.float32),
        scratch_types=[
            pltpu.VMEM((_CCH,), jnp.int32), pltpu.VMEM((_CCH,), jnp.int32),
            pltpu.VMEM((_CCH, H), jnp.float32),
            pltpu.VMEM((_CCH, H), jnp.float32),
            pltpu.VMEM((_CCH, H), jnp.float32),
            pltpu.SemaphoreType.DMA, pltpu.SemaphoreType.DMA,
        ],
    )(ys, pos0, pos1)


def kernel(x, Wg, W1, W2, W3):
    W1T = jnp.transpose(W1, (0, 2, 1)).astype(jnp.bfloat16)
    W3T = jnp.transpose(W3, (0, 2, 1)).astype(jnp.bfloat16)
    W2T = jnp.transpose(W2, (0, 2, 1)).astype(jnp.bfloat16)
    xb32 = lax.bitcast_convert_type(
        x.astype(jnp.bfloat16).reshape(T, H2, 2), jnp.int32)

    e1, e2, r1, r2, wa, wb, cnt = _router(x, Wg)
    counts = cnt[0].astype(jnp.int32)
    cap = ((counts + (BLK - 1)) // BLK) * BLK
    inc = jnp.cumsum(cap)
    off = (inc - cap).astype(jnp.int32)
    off16 = jnp.concatenate([off, jnp.zeros((8,), jnp.int32)])
    bvec = jnp.arange(NB, dtype=jnp.int32) * BLK
    be = jnp.minimum(
        jnp.sum((inc[None, :] <= bvec[:, None]).astype(jnp.int32), axis=1),
        E - 1).astype(jnp.int32)

    st, cwp, pos0, pos1 = _scatter(
        e1.reshape(T), e2.reshape(T), r1.reshape(T), r2.reshape(T),
        wa.reshape(T), wb.reshape(T), off16)
    xs32 = _gather(xb32, st)
    xs = lax.bitcast_convert_type(xs32, jnp.bfloat16).reshape(P, H)
    ys = _ffn(xs, be, W1T, W3T, W2T, cwp.reshape(P, 1))
    return _combine(ys, pos0, pos1)


# speedup vs baseline: 1.6943x; 1.6943x over previous
import jax
import jax.numpy as jnp
from jax import lax
from jax.experimental import pallas as pl
from jax.experimental.pallas import tpu as pltpu

H = 1024
F = 2048
E = 8
K = 2
T = 2048

TB = 256
NTB = T // TB
BLK = 256
P = 6144
NB = P // BLK


def _router_body(x_ref, wgt_ref, e1_ref, e2_ref, r1_ref, r2_ref,
                 wa_ref, wb_ref, cnt_ref, carry_ref):
    tb = pl.program_id(0)
    logits = lax.dot_general(
        x_ref[...].astype(jnp.bfloat16), wgt_ref[...].astype(jnp.bfloat16),
        (((1,), (0,)), ((), ())),
        preferred_element_type=jnp.float32)
    lane = lax.broadcasted_iota(jnp.int32, (TB, E), 1)
    big = jnp.int32(E)
    l1 = jnp.max(logits, axis=1, keepdims=True)
    i1 = jnp.min(jnp.where(logits == l1, lane, big), axis=1, keepdims=True)
    masked = jnp.where(lane == i1, -jnp.inf, logits)
    l2 = jnp.max(masked, axis=1, keepdims=True)
    i2 = jnp.min(jnp.where(masked == l2, lane, big), axis=1, keepdims=True)
    wb = 1.0 / (1.0 + jnp.exp(l1 - l2))
    wa = 1.0 - wb

    mask = ((lane == i1) | (lane == i2)).astype(jnp.bfloat16)
    row_i = lax.broadcasted_iota(jnp.int32, (TB, TB), 0)
    col_i = lax.broadcasted_iota(jnp.int32, (TB, TB), 1)
    tri = (col_i < row_i).astype(jnp.bfloat16)
    rank = lax.dot_general(tri, mask, (((1,), (0,)), ((), ())),
                           preferred_element_type=jnp.float32)

    @pl.when(tb == 0)
    def _():
        carry_ref[...] = jnp.zeros_like(carry_ref)

    carry = carry_ref[0:1, 0:E]
    rank = rank + carry
    new_carry = carry + jnp.sum(mask.astype(jnp.float32), axis=0,
                                keepdims=True)
    carry_ref[0:1, 0:E] = new_carry

    e1_ref[...] = i1
    e2_ref[...] = i2
    r1_ref[...] = jnp.sum(jnp.where(lane == i1, rank, 0.0), axis=1,
                          keepdims=True).astype(jnp.int32)
    r2_ref[...] = jnp.sum(jnp.where(lane == i2, rank, 0.0), axis=1,
                          keepdims=True).astype(jnp.int32)
    wa_ref[...] = wa
    wb_ref[...] = wb

    @pl.when(tb == NTB - 1)
    def _():
        cnt_ref[...] = jnp.broadcast_to(new_carry, (E, E))


def _router(x, Wg):
    out_shapes = [
        jax.ShapeDtypeStruct((T, 1), jnp.int32),
        jax.ShapeDtypeStruct((T, 1), jnp.int32),
        jax.ShapeDtypeStruct((T, 1), jnp.int32),
        jax.ShapeDtypeStruct((T, 1), jnp.int32),
        jax.ShapeDtypeStruct((T, 1), jnp.float32),
        jax.ShapeDtypeStruct((T, 1), jnp.float32),
        jax.ShapeDtypeStruct((E, E), jnp.float32),
    ]
    tok_spec = lambda: pl.BlockSpec((TB, 1), lambda tb: (tb, 0))
    return pl.pallas_call(
        _router_body,
        grid=(NTB,),
        in_specs=[
            pl.BlockSpec((TB, H), lambda tb: (tb, 0)),
            pl.BlockSpec((H, E), lambda tb: (0, 0)),
        ],
        out_specs=[
            tok_spec(), tok_spec(), tok_spec(), tok_spec(),
            tok_spec(), tok_spec(),
            pl.BlockSpec((E, E), lambda tb: (0, 0)),
        ],
        out_shape=out_shapes,
        scratch_shapes=[pltpu.VMEM((8, 128), jnp.float32)],
    )(x, Wg.T)


def _ffn_body(be_ref, xb_ref, w1_ref, w3_ref, w2_ref,
              p0r_ref, p1r_ref, p0c_ref, p1c_ref, wac_ref, wbc_ref,
              out_ref):
    b = pl.program_id(0)
    base = b * BLK

    rr = lax.broadcasted_iota(jnp.int32, (BLK, T), 0) + base
    gmask = ((p0r_ref[...] == rr) | (p1r_ref[...] == rr)).astype(jnp.bfloat16)
    xs = jnp.dot(gmask, xb_ref[...],
                 preferred_element_type=jnp.float32).astype(jnp.bfloat16)

    h = jnp.dot(xs, w1_ref[0], preferred_element_type=jnp.float32)
    g = jnp.dot(xs, w3_ref[0], preferred_element_type=jnp.float32)
    a = (h * lax.logistic(h) * g).astype(jnp.bfloat16)
    y = jnp.dot(a, w2_ref[0],
                preferred_element_type=jnp.float32).astype(jnp.bfloat16)

    rc = lax.broadcasted_iota(jnp.int32, (T, BLK), 1) + base
    sm = (jnp.where(p0c_ref[...] == rc, wac_ref[...], 0.0)
          + jnp.where(p1c_ref[...] == rc, wbc_ref[...], 0.0)).astype(
              jnp.bfloat16)
    contrib = jnp.dot(sm, y, preferred_element_type=jnp.float32)

    @pl.when(b == 0)
    def _():
        out_ref[...] = contrib

    @pl.when(b > 0)
    def _():
        out_ref[...] += contrib


def _ffn(xb, be, W1T, W3T, W2T, p0r, p1r, p0c, p1c, wac, wbc):
    grid_spec = pltpu.PrefetchScalarGridSpec(
        num_scalar_prefetch=1,
        grid=(NB,),
        in_specs=[
            pl.BlockSpec((T, H), lambda b, be: (0, 0)),
            pl.BlockSpec((1, H, F), lambda b, be: (be[b], 0, 0)),
            pl.BlockSpec((1, H, F), lambda b, be: (be[b], 0, 0)),
            pl.BlockSpec((1, F, H), lambda b, be: (be[b], 0, 0)),
            pl.BlockSpec((1, T), lambda b, be: (0, 0)),
            pl.BlockSpec((1, T), lambda b, be: (0, 0)),
            pl.BlockSpec((T, 1), lambda b, be: (0, 0)),
            pl.BlockSpec((T, 1), lambda b, be: (0, 0)),
            pl.BlockSpec((T, 1), lambda b, be: (0, 0)),
            pl.BlockSpec((T, 1), lambda b, be: (0, 0)),
        ],
        out_specs=pl.BlockSpec((T, H), lambda b, be: (0, 0)),
    )
    return pl.pallas_call(
        _ffn_body,
        grid_spec=grid_spec,
        out_shape=jax.ShapeDtypeStruct((T, H), jnp.float32),
        compiler_params=pltpu.CompilerParams(
            dimension_semantics=("arbitrary",)),
    )(be, xb, W1T, W3T, W2T, p0r, p1r, p0c, p1c, wac, wbc)


def kernel(x, Wg, W1, W2, W3):
    W1T = jnp.transpose(W1, (0, 2, 1)).astype(jnp.bfloat16)
    W3T = jnp.transpose(W3, (0, 2, 1)).astype(jnp.bfloat16)
    W2T = jnp.transpose(W2, (0, 2, 1)).astype(jnp.bfloat16)
    xb = x.astype(jnp.bfloat16)

    e1, e2, r1, r2, wa, wb, cnt = _router(x, Wg)
    counts = cnt[0].astype(jnp.int32)
    cap = ((counts + (BLK - 1)) // BLK) * BLK
    inc = jnp.cumsum(cap)
    off = (inc - cap).astype(jnp.int32)
    bvec = jnp.arange(NB, dtype=jnp.int32) * BLK
    be = jnp.minimum(
        jnp.sum((inc[None, :] <= bvec[:, None]).astype(jnp.int32), axis=1),
        E - 1).astype(jnp.int32)

    lane = jnp.arange(E, dtype=jnp.int32)[None, :]
    sel1 = (e1 == lane).astype(jnp.int32)
    sel2 = (e2 == lane).astype(jnp.int32)
    p0c = jnp.sum(sel1 * off[None, :], axis=1, keepdims=True) + r1
    p1c = jnp.sum(sel2 * off[None, :], axis=1, keepdims=True) + r2

    return _ffn(xb, be, W1T, W3T, W2T,
                p0c.reshape(1, T), p1c.reshape(1, T),
                p0c, p1c, wa, wb)


# speedup vs baseline: 2.0621x; 1.2171x over previous
import jax
import jax.numpy as jnp
from jax import lax
from jax.experimental import pallas as pl
from jax.experimental.pallas import tpu as pltpu

H = 1024
F = 2048
E = 8
K = 2
T = 2048

TB = 256
NTB = T // TB
BLK = 256
P = 6144
NB = P // BLK


def _router_body(x_ref, wgt_ref, e1_ref, e2_ref, r1_ref, r2_ref,
                 wa_ref, wb_ref, cnt_ref, carry_ref):
    tb = pl.program_id(0)
    logits = lax.dot_general(
        x_ref[...].astype(jnp.bfloat16), wgt_ref[...].astype(jnp.bfloat16),
        (((1,), (0,)), ((), ())),
        preferred_element_type=jnp.float32)
    lane = lax.broadcasted_iota(jnp.int32, (TB, E), 1)
    big = jnp.int32(E)
    l1 = jnp.max(logits, axis=1, keepdims=True)
    i1 = jnp.min(jnp.where(logits == l1, lane, big), axis=1, keepdims=True)
    masked = jnp.where(lane == i1, -jnp.inf, logits)
    l2 = jnp.max(masked, axis=1, keepdims=True)
    i2 = jnp.min(jnp.where(masked == l2, lane, big), axis=1, keepdims=True)
    wb = 1.0 / (1.0 + jnp.exp(l1 - l2))
    wa = 1.0 - wb

    mask = ((lane == i1) | (lane == i2)).astype(jnp.bfloat16)
    row_i = lax.broadcasted_iota(jnp.int32, (TB, TB), 0)
    col_i = lax.broadcasted_iota(jnp.int32, (TB, TB), 1)
    tri = (col_i < row_i).astype(jnp.bfloat16)
    rank = lax.dot_general(tri, mask, (((1,), (0,)), ((), ())),
                           preferred_element_type=jnp.float32)

    @pl.when(tb == 0)
    def _():
        carry_ref[...] = jnp.zeros_like(carry_ref)

    carry = carry_ref[0:1, 0:E]
    rank = rank + carry
    new_carry = carry + jnp.sum(mask.astype(jnp.float32), axis=0,
                                keepdims=True)
    carry_ref[0:1, 0:E] = new_carry

    e1_ref[...] = i1
    e2_ref[...] = i2
    r1_ref[...] = jnp.sum(jnp.where(lane == i1, rank, 0.0), axis=1,
                          keepdims=True).astype(jnp.int32)
    r2_ref[...] = jnp.sum(jnp.where(lane == i2, rank, 0.0), axis=1,
                          keepdims=True).astype(jnp.int32)
    wa_ref[...] = wa
    wb_ref[...] = wb

    @pl.when(tb == NTB - 1)
    def _():
        cnt_ref[...] = jnp.broadcast_to(new_carry, (E, E))


def _router(x, Wg):
    out_shapes = [
        jax.ShapeDtypeStruct((T, 1), jnp.int32),
        jax.ShapeDtypeStruct((T, 1), jnp.int32),
        jax.ShapeDtypeStruct((T, 1), jnp.int32),
        jax.ShapeDtypeStruct((T, 1), jnp.int32),
        jax.ShapeDtypeStruct((T, 1), jnp.float32),
        jax.ShapeDtypeStruct((T, 1), jnp.float32),
        jax.ShapeDtypeStruct((E, E), jnp.float32),
    ]
    tok_spec = lambda: pl.BlockSpec((TB, 1), lambda tb: (tb, 0))
    return pl.pallas_call(
        _router_body,
        grid=(NTB,),
        in_specs=[
            pl.BlockSpec((TB, H), lambda tb: (tb, 0)),
            pl.BlockSpec((H, E), lambda tb: (0, 0)),
        ],
        out_specs=[
            tok_spec(), tok_spec(), tok_spec(), tok_spec(),
            tok_spec(), tok_spec(),
            pl.BlockSpec((E, E), lambda tb: (0, 0)),
        ],
        out_shape=out_shapes,
        scratch_shapes=[pltpu.VMEM((8, 128), jnp.float32)],
    )(x, Wg.T)


def _ffn_body(be_ref, xb_ref, w1_ref, w3_ref, w2_ref,
              p0r_ref, p1r_ref, p0c_ref, p1c_ref, wac_ref, wbc_ref,
              out_ref):
    b = pl.program_id(0)
    base = b * BLK

    rr = lax.broadcasted_iota(jnp.int32, (BLK, T), 0) + base
    gmask = ((p0r_ref[...] == rr) | (p1r_ref[...] == rr)).astype(jnp.bfloat16)
    xs = jnp.dot(gmask, xb_ref[...],
                 preferred_element_type=jnp.float32).astype(jnp.bfloat16)

    h = lax.dot_general(xs, w1_ref[0], (((1,), (1,)), ((), ())),
                        preferred_element_type=jnp.float32)
    g = lax.dot_general(xs, w3_ref[0], (((1,), (1,)), ((), ())),
                        preferred_element_type=jnp.float32)
    a = (h * lax.logistic(h) * g).astype(jnp.bfloat16)
    y = lax.dot_general(a, w2_ref[0], (((1,), (1,)), ((), ())),
                        preferred_element_type=jnp.float32).astype(jnp.bfloat16)

    rc = lax.broadcasted_iota(jnp.int32, (T, BLK), 1) + base
    sm = (jnp.where(p0c_ref[...] == rc, wac_ref[...], 0.0)
          + jnp.where(p1c_ref[...] == rc, wbc_ref[...], 0.0)).astype(
              jnp.bfloat16)
    contrib = jnp.dot(sm, y, preferred_element_type=jnp.float32)

    @pl.when(b == 0)
    def _():
        out_ref[...] = contrib

    @pl.when(b > 0)
    def _():
        out_ref[...] += contrib


def _ffn(xb, be, W1T, W3T, W2T, p0r, p1r, p0c, p1c, wac, wbc):
    grid_spec = pltpu.PrefetchScalarGridSpec(
        num_scalar_prefetch=1,
        grid=(NB,),
        in_specs=[
            pl.BlockSpec((T, H), lambda b, be: (0, 0)),
            pl.BlockSpec((1, F, H), lambda b, be: (be[b], 0, 0)),
            pl.BlockSpec((1, F, H), lambda b, be: (be[b], 0, 0)),
            pl.BlockSpec((1, H, F), lambda b, be: (be[b], 0, 0)),
            pl.BlockSpec((1, T), lambda b, be: (0, 0)),
            pl.BlockSpec((1, T), lambda b, be: (0, 0)),
            pl.BlockSpec((T, 1), lambda b, be: (0, 0)),
            pl.BlockSpec((T, 1), lambda b, be: (0, 0)),
            pl.BlockSpec((T, 1), lambda b, be: (0, 0)),
            pl.BlockSpec((T, 1), lambda b, be: (0, 0)),
        ],
        out_specs=pl.BlockSpec((T, H), lambda b, be: (0, 0)),
    )
    return pl.pallas_call(
        _ffn_body,
        grid_spec=grid_spec,
        out_shape=jax.ShapeDtypeStruct((T, H), jnp.float32),
        compiler_params=pltpu.CompilerParams(
            dimension_semantics=("arbitrary",)),
    )(be, xb, W1T, W3T, W2T, p0r, p1r, p0c, p1c, wac, wbc)


def kernel(x, Wg, W1, W2, W3):
    W1b = W1.astype(jnp.bfloat16)
    W3b = W3.astype(jnp.bfloat16)
    W2b = W2.astype(jnp.bfloat16)
    xb = x.astype(jnp.bfloat16)

    e1, e2, r1, r2, wa, wb, cnt = _router(x, Wg)
    counts = cnt[0].astype(jnp.int32)
    cap = ((counts + (BLK - 1)) // BLK) * BLK
    inc = jnp.cumsum(cap)
    off = (inc - cap).astype(jnp.int32)
    bvec = jnp.arange(NB, dtype=jnp.int32) * BLK
    be = jnp.minimum(
        jnp.sum((inc[None, :] <= bvec[:, None]).astype(jnp.int32), axis=1),
        E - 1).astype(jnp.int32)

    lane = jnp.arange(E, dtype=jnp.int32)[None, :]
    sel1 = (e1 == lane).astype(jnp.int32)
    sel2 = (e2 == lane).astype(jnp.int32)
    p0c = jnp.sum(sel1 * off[None, :], axis=1, keepdims=True) + r1
    p1c = jnp.sum(sel2 * off[None, :], axis=1, keepdims=True) + r2

    return _ffn(xb, be, W1b, W3b, W2b,
                p0c.reshape(1, T), p1c.reshape(1, T),
                p0c, p1c, wa, wb)


# speedup vs baseline: 2.2766x; 1.1040x over previous
import jax
import jax.numpy as jnp
from jax import lax
from jax.experimental import pallas as pl
from jax.experimental.pallas import tpu as pltpu

H = 1024
F = 2048
E = 8
K = 2
T = 2048

TB = 256
NTB = T // TB
BLK = 256
P = 6144
NB = P // BLK


def _router_body(x_ref, wgt_ref, e1_ref, e2_ref, r1_ref, r2_ref,
                 wa_ref, wb_ref, cnt_ref, carry_ref):
    tb = pl.program_id(0)
    logits = lax.dot_general(
        x_ref[...].astype(jnp.bfloat16), wgt_ref[...].astype(jnp.bfloat16),
        (((1,), (0,)), ((), ())),
        preferred_element_type=jnp.float32)
    lane = lax.broadcasted_iota(jnp.int32, (TB, E), 1)
    big = jnp.int32(E)
    l1 = jnp.max(logits, axis=1, keepdims=True)
    i1 = jnp.min(jnp.where(logits == l1, lane, big), axis=1, keepdims=True)
    masked = jnp.where(lane == i1, -jnp.inf, logits)
    l2 = jnp.max(masked, axis=1, keepdims=True)
    i2 = jnp.min(jnp.where(masked == l2, lane, big), axis=1, keepdims=True)
    wb = 1.0 / (1.0 + jnp.exp(l1 - l2))
    wa = 1.0 - wb

    mask = ((lane == i1) | (lane == i2)).astype(jnp.bfloat16)
    row_i = lax.broadcasted_iota(jnp.int32, (TB, TB), 0)
    col_i = lax.broadcasted_iota(jnp.int32, (TB, TB), 1)
    tri = (col_i < row_i).astype(jnp.bfloat16)
    rank = lax.dot_general(tri, mask, (((1,), (0,)), ((), ())),
                           preferred_element_type=jnp.float32)

    @pl.when(tb == 0)
    def _():
        carry_ref[...] = jnp.zeros_like(carry_ref)

    carry = carry_ref[0:1, 0:E]
    rank = rank + carry
    new_carry = carry + jnp.sum(mask.astype(jnp.float32), axis=0,
                                keepdims=True)
    carry_ref[0:1, 0:E] = new_carry

    e1_ref[...] = i1
    e2_ref[...] = i2
    r1_ref[...] = jnp.sum(jnp.where(lane == i1, rank, 0.0), axis=1,
                          keepdims=True).astype(jnp.int32)
    r2_ref[...] = jnp.sum(jnp.where(lane == i2, rank, 0.0), axis=1,
                          keepdims=True).astype(jnp.int32)
    wa_ref[...] = wa
    wb_ref[...] = wb

    @pl.when(tb == NTB - 1)
    def _():
        cnt_ref[...] = jnp.broadcast_to(new_carry, (E, E))


def _router(x, Wg):
    out_shapes = [
        jax.ShapeDtypeStruct((T, 1), jnp.int32),
        jax.ShapeDtypeStruct((T, 1), jnp.int32),
        jax.ShapeDtypeStruct((T, 1), jnp.int32),
        jax.ShapeDtypeStruct((T, 1), jnp.int32),
        jax.ShapeDtypeStruct((T, 1), jnp.float32),
        jax.ShapeDtypeStruct((T, 1), jnp.float32),
        jax.ShapeDtypeStruct((E, E), jnp.float32),
    ]
    tok_spec = lambda: pl.BlockSpec((TB, 1), lambda tb: (tb, 0))
    return pl.pallas_call(
        _router_body,
        grid=(NTB,),
        in_specs=[
            pl.BlockSpec((TB, H), lambda tb: (tb, 0)),
            pl.BlockSpec((H, E), lambda tb: (0, 0)),
        ],
        out_specs=[
            tok_spec(), tok_spec(), tok_spec(), tok_spec(),
            tok_spec(), tok_spec(),
            pl.BlockSpec((E, E), lambda tb: (0, 0)),
        ],
        out_shape=out_shapes,
        scratch_shapes=[pltpu.VMEM((8, 128), jnp.float32)],
    )(x, Wg.T)


FH = F // 2


def _ffn_body(be_ref, xb_ref, w1_ref, w3_ref, w2_ref,
              p0r_ref, p1r_ref, p0c_ref, p1c_ref, wac_ref, wbc_ref,
              out_ref, xs_s, y_s):
    b = pl.program_id(0)
    fh = pl.program_id(1)
    base = b * BLK

    @pl.when(fh == 0)
    def _():
        rr = lax.broadcasted_iota(jnp.int32, (BLK, T), 0) + base
        gmask = ((p0r_ref[...] == rr)
                 | (p1r_ref[...] == rr)).astype(jnp.bfloat16)
        xs_s[...] = jnp.dot(gmask, xb_ref[...],
                            preferred_element_type=jnp.float32).astype(
                                jnp.bfloat16)

    xs = xs_s[...]
    h = lax.dot_general(xs, w1_ref[0], (((1,), (1,)), ((), ())),
                        preferred_element_type=jnp.float32)
    g = lax.dot_general(xs, w3_ref[0], (((1,), (1,)), ((), ())),
                        preferred_element_type=jnp.float32)
    a = (h * lax.logistic(h) * g).astype(jnp.bfloat16)
    ypart = lax.dot_general(a, w2_ref[0], (((1,), (1,)), ((), ())),
                            preferred_element_type=jnp.float32)

    @pl.when(fh == 0)
    def _():
        y_s[...] = ypart

    @pl.when(fh == 1)
    def _():
        y = (y_s[...] + ypart).astype(jnp.bfloat16)
        rc = lax.broadcasted_iota(jnp.int32, (T, BLK), 1) + base
        sm = (jnp.where(p0c_ref[...] == rc, wac_ref[...], 0.0)
              + jnp.where(p1c_ref[...] == rc, wbc_ref[...], 0.0)).astype(
                  jnp.bfloat16)
        contrib = jnp.dot(sm, y, preferred_element_type=jnp.float32)

        @pl.when(b == 0)
        def _():
            out_ref[...] = contrib

        @pl.when(b > 0)
        def _():
            out_ref[...] += contrib


def _ffn(xb, be, W1, W3, W2, p0r, p1r, p0c, p1c, wac, wbc):
    grid_spec = pltpu.PrefetchScalarGridSpec(
        num_scalar_prefetch=1,
        grid=(NB, 2),
        in_specs=[
            pl.BlockSpec((T, H), lambda b, fh, be: (0, 0)),
            pl.BlockSpec((1, FH, H), lambda b, fh, be: (be[b], fh, 0)),
            pl.BlockSpec((1, FH, H), lambda b, fh, be: (be[b], fh, 0)),
            pl.BlockSpec((1, H, FH), lambda b, fh, be: (be[b], 0, fh)),
            pl.BlockSpec((1, T), lambda b, fh, be: (0, 0)),
            pl.BlockSpec((1, T), lambda b, fh, be: (0, 0)),
            pl.BlockSpec((T, 1), lambda b, fh, be: (0, 0)),
            pl.BlockSpec((T, 1), lambda b, fh, be: (0, 0)),
            pl.BlockSpec((T, 1), lambda b, fh, be: (0, 0)),
            pl.BlockSpec((T, 1), lambda b, fh, be: (0, 0)),
        ],
        out_specs=pl.BlockSpec((T, H), lambda b, fh, be: (0, 0)),
        scratch_shapes=[
            pltpu.VMEM((BLK, H), jnp.bfloat16),
            pltpu.VMEM((BLK, H), jnp.float32),
        ],
    )
    return pl.pallas_call(
        _ffn_body,
        grid_spec=grid_spec,
        out_shape=jax.ShapeDtypeStruct((T, H), jnp.float32),
        compiler_params=pltpu.CompilerParams(
            dimension_semantics=("arbitrary", "arbitrary")),
    )(be, xb, W1, W3, W2, p0r, p1r, p0c, p1c, wac, wbc)


def kernel(x, Wg, W1, W2, W3):
    xb = x.astype(jnp.bfloat16)

    e1, e2, r1, r2, wa, wb, cnt = _router(x, Wg)
    counts = cnt[0].astype(jnp.int32)
    cap = ((counts + (BLK - 1)) // BLK) * BLK
    inc = jnp.cumsum(cap)
    off = (inc - cap).astype(jnp.int32)
    bvec = jnp.arange(NB, dtype=jnp.int32) * BLK
    be = jnp.minimum(
        jnp.sum((inc[None, :] <= bvec[:, None]).astype(jnp.int32), axis=1),
        E - 1).astype(jnp.int32)

    lane = jnp.arange(E, dtype=jnp.int32)[None, :]
    sel1 = (e1 == lane).astype(jnp.int32)
    sel2 = (e2 == lane).astype(jnp.int32)
    p0c = jnp.sum(sel1 * off[None, :], axis=1, keepdims=True) + r1
    p1c = jnp.sum(sel2 * off[None, :], axis=1, keepdims=True) + r2

    return _ffn(xb, be, W1, W3, W2,
                p0c.reshape(1, T), p1c.reshape(1, T),
                p0c, p1c, wa, wb)
